# baseline (speedup 1.0000x reference)
import functools

import jax
import jax.numpy as jnp
from jax import lax
from jax.experimental import pallas as pl
from jax.experimental.pallas import tpu as pltpu
from jax.experimental.pallas import tpu_sc as plsc

NSEG = 4096
NCORE = 2
NSUB = 16
NW = NCORE * NSUB
LANES = 16
WIN = 4000


def _iota():
    return lax.iota(jnp.int32, LANES)


def _phase1_body(n_pts, pts_hbm, batch_hbm, part_hbm, pts_win, batch_win,
                 acc, red, shared):
    cid = lax.axis_index("c")
    sid = lax.axis_index("s")
    wid = sid * NCORE + cid
    pts_per = n_pts // NW
    nwin = pts_per // WIN

    iota = _iota()
    q4 = lax.shift_right_logical(iota, 2)
    zero16 = jnp.zeros((LANES,), jnp.float32)

    def zero_body(i, _):
        acc[pl.ds(i * LANES, LANES)] = zero16
        return 0
    lax.fori_loop(0, NSEG, zero_body, 0)

    def win_body(w, _):
        base = wid * pts_per + w * WIN
        pltpu.sync_copy(pts_hbm.at[pl.ds(4 * base, 4 * WIN)],
                        pts_win.at[pl.ds(0, 4 * WIN)])
        pltpu.sync_copy(batch_hbm.at[pl.ds(base, WIN)], batch_win)

        def grp_body(i, _):
            for u in range(4):
                g = i * 4 + u
                ids4 = plsc.load_gather(batch_win, [q4 + 4 * g])
                addr = ids4 * LANES + iota
                vals = pts_win[pl.ds(16 * g, LANES)]
                plsc.addupdate_scatter(acc, [addr], vals)
            return 0
        lax.fori_loop(0, WIN // LANES, grp_body, 0)
        return 0
    lax.fori_loop(0, nwin, win_body, 0)

    ix8 = iota ^ 8
    ix4 = iota ^ 4
    ix12 = iota ^ 12
    mlow = iota < 4

    def red_body(s, _):
        b = s * LANES
        v = acc[pl.ds(b, LANES)]
        g8 = plsc.load_gather(acc, [b + ix8])
        g4 = plsc.load_gather(acc, [b + ix4])
        g12 = plsc.load_gather(acc, [b + ix12])
        v4 = (v + g8) + (g4 + g12)
        plsc.store_scatter(red, [4 * s + iota], v4, mask=mlow)
        return 0
    lax.fori_loop(0, NSEG, red_body, 0)

    pltpu.sync_copy(red.at[pl.ds(0, 4 * NSEG)], shared.at[sid])
    plsc.subcore_barrier()

    seg = 4 * NSEG // NSUB
    for r in range(NSUB):
        pltpu.sync_copy(shared.at[r, pl.ds(sid * seg, seg)],
                        acc.at[pl.ds(r * seg, seg)])

    def sum_body(j, _):
        t = zero16
        for r in range(NSUB):
            t = t + acc[pl.ds(r * seg + j * LANES, LANES)]
        red[pl.ds(j * LANES, LANES)] = t
        return 0
    lax.fori_loop(0, seg // LANES, sum_body, 0)

    pltpu.sync_copy(red.at[pl.ds(0, seg)],
                    part_hbm.at[pl.ds(cid * 4 * NSEG + sid * seg, seg)])


def _phase2_body(n_pts, pts_hbm, batch_hbm, part_hbm, par_hbm, out_hbm,
                 pts_win, batch_win, out_win, buf_a, buf_b, hbuf, pbuf):
    cid = lax.axis_index("c")
    sid = lax.axis_index("s")
    wid = sid * NCORE + cid
    pts_per = n_pts // NW
    nwin = pts_per // WIN

    iota = _iota()
    r4 = lax.bitwise_and(iota, 3)
    mask3 = r4 == 3
    or3 = iota | 3
    ix1 = iota ^ 1
    ix2 = iota ^ 2
    ix3 = iota ^ 3
    iota4 = 4 * iota

    pltpu.sync_copy(part_hbm.at[pl.ds(0, 4 * NSEG)], buf_a)
    pltpu.sync_copy(part_hbm.at[pl.ds(4 * NSEG, 4 * NSEG)], buf_b)
    pltpu.sync_copy(par_hbm, pbuf)
    av = pbuf[pl.ds(0, LANES)]
    c0v = pbuf[pl.ds(LANES, LANES)]

    def h_body(i, _):
        b = i * LANES
        v = buf_a[pl.ds(b, LANES)] + buf_b[pl.ds(b, LANES)]
        buf_a[pl.ds(b, LANES)] = v
        cnt = plsc.load_gather(buf_a, [b + or3])
        c = v / jnp.maximum(cnt, 1.0)
        csq = jnp.where(mask3, 0.0, c * c)
        buf_b[pl.ds(b, LANES)] = csq
        s2 = ((csq + plsc.load_gather(buf_b, [b + ix1]))
              + (plsc.load_gather(buf_b, [b + ix2])
                 + plsc.load_gather(buf_b, [b + ix3])))
        h = jnp.where(mask3, av * s2 + c0v, (-2.0 * av) * c)
        hbuf[pl.ds(b, LANES)] = h
        return 0
    lax.fori_loop(0, 4 * NSEG // LANES, h_body, 0)

    def win_body(w, _):
        base = wid * pts_per + w * WIN
        pltpu.sync_copy(pts_hbm.at[pl.ds(4 * base, 4 * WIN)],
                        pts_win.at[pl.ds(0, 4 * WIN)])
        pltpu.sync_copy(batch_hbm.at[pl.ds(base, WIN)], batch_win)

        def pt_body(i, _):
            off = i * LANES
            idv = batch_win[pl.ds(off, LANES)]
            pidx = 64 * i + iota4
            xs = plsc.load_gather(pts_win, [pidx])
            ys = plsc.load_gather(pts_win, [pidx + 1])
            zs = plsc.load_gather(pts_win, [pidx + 2])
            hidx = idv * 4
            hx = plsc.load_gather(hbuf, [hidx])
            hy = plsc.load_gather(hbuf, [hidx + 1])
            hz = plsc.load_gather(hbuf, [hidx + 2])
            hw = plsc.load_gather(hbuf, [hidx + 3])
            s2p = xs * xs + ys * ys + zs * zs
            dot = xs * hx + ys * hy + zs * hz
            logit = av * s2p + dot + hw
            sig = 1.0 / (1.0 + jnp.exp(-logit))
            out_win[pl.ds(off, LANES)] = sig + 1e-4
            return 0
        lax.fori_loop(0, WIN // LANES, pt_body, 0)

        pltpu.sync_copy(out_win, out_hbm.at[pl.ds(base, WIN)])
        return 0
    lax.fori_loop(0, nwin, win_body, 0)


def kernel(pos, batch, W, b):
    n = pos.shape[0]
    pts4 = jnp.concatenate(
        [pos, jnp.ones((n, 1), jnp.float32)], axis=1).reshape(-1)
    w00 = jnp.full((LANES,), W[0, 0], jnp.float32)
    c0 = jnp.full((LANES,), W[0, 1] + b[0], jnp.float32)
    params = jnp.concatenate([w00, c0])

    mesh = plsc.VectorSubcoreMesh(core_axis_name="c", subcore_axis_name="s")
    cparams = pltpu.CompilerParams(needs_layout_passes=False)

    k1 = functools.partial(
        pl.kernel,
        out_type=jax.ShapeDtypeStruct((NCORE * 4 * NSEG,), jnp.float32),
        mesh=mesh,
        compiler_params=cparams,
        scratch_types=[
            pltpu.VMEM((4 * WIN,), jnp.float32),
            pltpu.VMEM((WIN,), jnp.int32),
            pltpu.VMEM((NSEG * LANES,), jnp.float32),
            pltpu.VMEM((4 * NSEG + 16,), jnp.float32),
            pltpu.VMEM_SHARED((NSUB, 4 * NSEG), jnp.float32),
        ],
    )(functools.partial(_phase1_body, n))
    part = k1(pts4, batch)

    k2 = functools.partial(
        pl.kernel,
        out_type=jax.ShapeDtypeStruct((n,), jnp.float32),
        mesh=mesh,
        compiler_params=cparams,
        scratch_types=[
            pltpu.VMEM((4 * WIN,), jnp.float32),
            pltpu.VMEM((WIN,), jnp.int32),
            pltpu.VMEM((WIN,), jnp.float32),
            pltpu.VMEM((4 * NSEG,), jnp.float32),
            pltpu.VMEM((4 * NSEG,), jnp.float32),
            pltpu.VMEM((4 * NSEG + 16,), jnp.float32),
            pltpu.VMEM((2 * LANES,), jnp.float32),
        ],
    )(functools.partial(_phase2_body, n))
    return k2(pts4, batch, part, params)


# speedup vs baseline: 1.0105x; 1.0105x over previous
import functools

import jax
import jax.numpy as jnp
from jax import lax
from jax.experimental import pallas as pl
from jax.experimental.pallas import tpu as pltpu
from jax.experimental.pallas import tpu_sc as plsc

NSEG = 4096
NCORE = 2
NSUB = 16
NW = NCORE * NSUB
LANES = 16
WIN = 2000


def _iota():
    return lax.iota(jnp.int32, LANES)


def _phase1_body(n_pts, pos_hbm, batch_hbm, part_hbm, pos_win, batch_win,
                 acc, red, shared):
    cid = lax.axis_index("c")
    sid = lax.axis_index("s")
    wid = sid * NCORE + cid
    pts_per = n_pts // NW
    nwin = pts_per // WIN

    iota = _iota()
    q4 = lax.shift_right_logical(iota, 2)
    r4 = lax.bitwise_and(iota, 3)
    mask3 = r4 == 3
    col3 = jnp.where(mask3, 0, r4)
    zero16 = jnp.zeros((LANES,), jnp.float32)

    def zero_body(i, _):
        acc[pl.ds(i * LANES, LANES)] = zero16
        return 0
    lax.fori_loop(0, NSEG, zero_body, 0)

    def win_body(w, _):
        base = wid * pts_per + w * WIN
        pltpu.sync_copy(pos_hbm.at[pl.ds(base, WIN)], pos_win)
        pltpu.sync_copy(batch_hbm.at[pl.ds(base, WIN)], batch_win)

        def grp_body(i, _):
            for u in range(4):
                g = i * 4 + u
                ids4 = plsc.load_gather(batch_win, [q4 + 4 * g])
                addr = ids4 * LANES + iota
                vals = plsc.load_gather(pos_win, [q4 + 4 * g, col3])
                vals = jnp.where(mask3, 1.0, vals)
                plsc.addupdate_scatter(acc, [addr], vals)
            return 0
        lax.fori_loop(0, WIN // LANES, grp_body, 0)
        return 0
    lax.fori_loop(0, nwin, win_body, 0)

    ix8 = iota ^ 8
    ix4 = iota ^ 4
    ix12 = iota ^ 12
    mlow = iota < 4

    def red_body(s, _):
        b = s * LANES
        v = acc[pl.ds(b, LANES)]
        g8 = plsc.load_gather(acc, [b + ix8])
        g4 = plsc.load_gather(acc, [b + ix4])
        g12 = plsc.load_gather(acc, [b + ix12])
        v4 = (v + g8) + (g4 + g12)
        plsc.store_scatter(red, [4 * s + iota], v4, mask=mlow)
        return 0
    lax.fori_loop(0, NSEG, red_body, 0)

    pltpu.sync_copy(red.at[pl.ds(0, 4 * NSEG)], shared.at[sid])
    plsc.subcore_barrier()

    seg = 4 * NSEG // NSUB
    for r in range(NSUB):
        pltpu.sync_copy(shared.at[r, pl.ds(sid * seg, seg)],
                        acc.at[pl.ds(r * seg, seg)])

    def sum_body(j, _):
        t = zero16
        for r in range(NSUB):
            t = t + acc[pl.ds(r * seg + j * LANES, LANES)]
        red[pl.ds(j * LANES, LANES)] = t
        return 0
    lax.fori_loop(0, seg // LANES, sum_body, 0)

    pltpu.sync_copy(red.at[pl.ds(0, seg)],
                    part_hbm.at[pl.ds(cid * 4 * NSEG + sid * seg, seg)])


def _phase2_body(n_pts, pos_hbm, batch_hbm, part_hbm, par_hbm, out_hbm,
                 pos_win, batch_win, out_win, buf_a, buf_b, hbuf, pbuf):
    cid = lax.axis_index("c")
    sid = lax.axis_index("s")
    wid = sid * NCORE + cid
    pts_per = n_pts // NW
    nwin = pts_per // WIN

    iota = _iota()
    r4 = lax.bitwise_and(iota, 3)
    mask3 = r4 == 3
    or3 = iota | 3
    ix1 = iota ^ 1
    ix2 = iota ^ 2
    ix3 = iota ^ 3
    zcol = jnp.zeros((LANES,), jnp.int32)

    pltpu.sync_copy(part_hbm.at[pl.ds(0, 4 * NSEG)], buf_a)
    pltpu.sync_copy(part_hbm.at[pl.ds(4 * NSEG, 4 * NSEG)], buf_b)
    pltpu.sync_copy(par_hbm, pbuf)
    av = pbuf[pl.ds(0, LANES)]
    c0v = pbuf[pl.ds(LANES, LANES)]

    def h_body(i, _):
        b = i * LANES
        v = buf_a[pl.ds(b, LANES)] + buf_b[pl.ds(b, LANES)]
        buf_a[pl.ds(b, LANES)] = v
        cnt = plsc.load_gather(buf_a, [b + or3])
        c = v / jnp.maximum(cnt, 1.0)
        csq = jnp.where(mask3, 0.0, c * c)
        buf_b[pl.ds(b, LANES)] = csq
        s2 = ((csq + plsc.load_gather(buf_b, [b + ix1]))
              + (plsc.load_gather(buf_b, [b + ix2])
                 + plsc.load_gather(buf_b, [b + ix3])))
        h = jnp.where(mask3, av * s2 + c0v, (-2.0 * av) * c)
        hbuf[pl.ds(b, LANES)] = h
        return 0
    lax.fori_loop(0, 4 * NSEG // LANES, h_body, 0)

    def win_body(w, _):
        base = wid * pts_per + w * WIN
        pltpu.sync_copy(pos_hbm.at[pl.ds(base, WIN)], pos_win)
        pltpu.sync_copy(batch_hbm.at[pl.ds(base, WIN)], batch_win)

        def pt_body(i, _):
            off = i * LANES
            idv = batch_win[pl.ds(off, LANES)]
            ridx = off + iota
            xs = plsc.load_gather(pos_win, [ridx, zcol])
            ys = plsc.load_gather(pos_win, [ridx, zcol + 1])
            zs = plsc.load_gather(pos_win, [ridx, zcol + 2])
            hidx = idv * 4
            hx = plsc.load_gather(hbuf, [hidx])
            hy = plsc.load_gather(hbuf, [hidx + 1])
            hz = plsc.load_gather(hbuf, [hidx + 2])
            hw = plsc.load_gather(hbuf, [hidx + 3])
            s2p = xs * xs + ys * ys + zs * zs
            dot = xs * hx + ys * hy + zs * hz
            logit = av * s2p + dot + hw
            sig = 1.0 / (1.0 + jnp.exp(-logit))
            out_win[pl.ds(off, LANES)] = sig + 1e-4
            return 0
        lax.fori_loop(0, WIN // LANES, pt_body, 0)

        pltpu.sync_copy(out_win, out_hbm.at[pl.ds(base, WIN)])
        return 0
    lax.fori_loop(0, nwin, win_body, 0)


def kernel(pos, batch, W, b):
    n = pos.shape[0]
    w00 = jnp.full((LANES,), W[0, 0], jnp.float32)
    c0 = jnp.full((LANES,), W[0, 1] + b[0], jnp.float32)
    params = jnp.concatenate([w00, c0])

    mesh = plsc.VectorSubcoreMesh(core_axis_name="c", subcore_axis_name="s")
    cparams = pltpu.CompilerParams(needs_layout_passes=False,
                                   use_tc_tiling_on_sc=False)

    k1 = functools.partial(
        pl.kernel,
        out_type=jax.ShapeDtypeStruct((NCORE * 4 * NSEG,), jnp.float32),
        mesh=mesh,
        compiler_params=cparams,
        scratch_types=[
            pltpu.VMEM((WIN, 3), jnp.float32),
            pltpu.VMEM((WIN,), jnp.int32),
            pltpu.VMEM((NSEG * LANES,), jnp.float32),
            pltpu.VMEM((4 * NSEG + 16,), jnp.float32),
            pltpu.VMEM_SHARED((NSUB, 4 * NSEG), jnp.float32),
        ],
    )(functools.partial(_phase1_body, n))
    part = k1(pos, batch)

    k2 = functools.partial(
        pl.kernel,
        out_type=jax.ShapeDtypeStruct((n,), jnp.float32),
        mesh=mesh,
        compiler_params=cparams,
        scratch_types=[
            pltpu.VMEM((WIN, 3), jnp.float32),
            pltpu.VMEM((WIN,), jnp.int32),
            pltpu.VMEM((WIN,), jnp.float32),
            pltpu.VMEM((4 * NSEG,), jnp.float32),
            pltpu.VMEM((4 * NSEG,), jnp.float32),
            pltpu.VMEM((4 * NSEG + 16,), jnp.float32),
            pltpu.VMEM((2 * LANES,), jnp.float32),
        ],
    )(functools.partial(_phase2_body, n))
    return k2(pos, batch, part, params)


# speedup vs baseline: 15.1963x; 15.0384x over previous
import functools

import jax
import jax.numpy as jnp
from jax import lax
from jax.experimental import pallas as pl
from jax.experimental.pallas import tpu as pltpu
from jax.experimental.pallas import tpu_sc as plsc

NSEG = 4096
NCORE = 2
NSUB = 16
NW = NCORE * NSUB
LANES = 16
WIN = 2000


def _iota():
    return lax.iota(jnp.int32, LANES)


def _phase1_body(n_pts, x_hbm, y_hbm, z_hbm, batch_hbm, part_hbm,
                 x_win, y_win, z_win, batch_win, acc, red, shared):
    cid = lax.axis_index("c")
    sid = lax.axis_index("s")
    wid = sid * NCORE + cid
    pts_per = n_pts // NW
    nwin = pts_per // WIN

    iota = _iota()
    q4 = lax.shift_right_logical(iota, 2)
    r4 = lax.bitwise_and(iota, 3)
    zero16 = jnp.zeros((LANES,), jnp.float32)

    def zero_body(i, _):
        acc[pl.ds(i * LANES, LANES)] = zero16
        return 0
    lax.fori_loop(0, NSEG, zero_body, 0)

    def win_body(w, _):
        base = wid * pts_per + w * WIN
        pltpu.sync_copy(x_hbm.at[pl.ds(base, WIN)], x_win)
        pltpu.sync_copy(y_hbm.at[pl.ds(base, WIN)], y_win)
        pltpu.sync_copy(z_hbm.at[pl.ds(base, WIN)], z_win)
        pltpu.sync_copy(batch_hbm.at[pl.ds(base, WIN)], batch_win)

        def grp_body(i, _):
            for u in range(4):
                g = i * 4 + u
                idx4 = q4 + 4 * g
                ids4 = plsc.load_gather(batch_win, [idx4])
                addr = ids4 * LANES + iota
                xg = plsc.load_gather(x_win, [idx4])
                yg = plsc.load_gather(y_win, [idx4])
                zg = plsc.load_gather(z_win, [idx4])
                vals = jnp.where(r4 == 0, xg,
                                 jnp.where(r4 == 1, yg,
                                           jnp.where(r4 == 2, zg, 1.0)))
                plsc.addupdate_scatter(acc, [addr], vals)
            return 0
        lax.fori_loop(0, WIN // LANES, grp_body, 0)
        return 0
    lax.fori_loop(0, nwin, win_body, 0)

    ix8 = iota ^ 8
    ix4 = iota ^ 4
    ix12 = iota ^ 12
    mlow = iota < 4

    def red_body(s, _):
        b = s * LANES
        v = acc[pl.ds(b, LANES)]
        g8 = plsc.load_gather(acc, [b + ix8])
        g4 = plsc.load_gather(acc, [b + ix4])
        g12 = plsc.load_gather(acc, [b + ix12])
        v4 = (v + g8) + (g4 + g12)
        plsc.store_scatter(red, [4 * s + iota], v4, mask=mlow)
        return 0
    lax.fori_loop(0, NSEG, red_body, 0)

    pltpu.sync_copy(red.at[pl.ds(0, 4 * NSEG)], shared.at[sid])
    plsc.subcore_barrier()

    seg = 4 * NSEG // NSUB
    for r in range(NSUB):
        pltpu.sync_copy(shared.at[r, pl.ds(sid * seg, seg)],
                        acc.at[pl.ds(r * seg, seg)])

    def sum_body(j, _):
        t = zero16
        for r in range(NSUB):
            t = t + acc[pl.ds(r * seg + j * LANES, LANES)]
        red[pl.ds(j * LANES, LANES)] = t
        return 0
    lax.fori_loop(0, seg // LANES, sum_body, 0)

    pltpu.sync_copy(red.at[pl.ds(0, seg)],
                    part_hbm.at[pl.ds(cid * 4 * NSEG + sid * seg, seg)])


def _phase2_body(n_pts, x_hbm, y_hbm, z_hbm, batch_hbm, part_hbm, par_hbm,
                 out_hbm, x_win, y_win, z_win, batch_win, out_win,
                 buf_a, buf_b, hbuf, pbuf):
    cid = lax.axis_index("c")
    sid = lax.axis_index("s")
    wid = sid * NCORE + cid
    pts_per = n_pts // NW
    nwin = pts_per // WIN

    iota = _iota()
    r4 = lax.bitwise_and(iota, 3)
    mask3 = r4 == 3
    or3 = iota | 3
    ix1 = iota ^ 1
    ix2 = iota ^ 2
    ix3 = iota ^ 3

    pltpu.sync_copy(part_hbm.at[pl.ds(0, 4 * NSEG)], buf_a)
    pltpu.sync_copy(part_hbm.at[pl.ds(4 * NSEG, 4 * NSEG)], buf_b)
    pltpu.sync_copy(par_hbm, pbuf)
    av = pbuf[pl.ds(0, LANES)]
    c0v = pbuf[pl.ds(LANES, LANES)]

    def h_body(i, _):
        b = i * LANES
        v = buf_a[pl.ds(b, LANES)] + buf_b[pl.ds(b, LANES)]
        buf_a[pl.ds(b, LANES)] = v
        cnt = plsc.load_gather(buf_a, [b + or3])
        c = v / jnp.maximum(cnt, 1.0)
        csq = jnp.where(mask3, 0.0, c * c)
        buf_b[pl.ds(b, LANES)] = csq
        s2 = ((csq + plsc.load_gather(buf_b, [b + ix1]))
              + (plsc.load_gather(buf_b, [b + ix2])
                 + plsc.load_gather(buf_b, [b + ix3])))
        h = jnp.where(mask3, av * s2 + c0v, (-2.0 * av) * c)
        hbuf[pl.ds(b, LANES)] = h
        return 0
    lax.fori_loop(0, 4 * NSEG // LANES, h_body, 0)

    def win_body(w, _):
        base = wid * pts_per + w * WIN
        pltpu.sync_copy(x_hbm.at[pl.ds(base, WIN)], x_win)
        pltpu.sync_copy(y_hbm.at[pl.ds(base, WIN)], y_win)
        pltpu.sync_copy(z_hbm.at[pl.ds(base, WIN)], z_win)
        pltpu.sync_copy(batch_hbm.at[pl.ds(base, WIN)], batch_win)

        def pt_body(i, _):
            off = i * LANES
            idv = batch_win[pl.ds(off, LANES)]
            xs = x_win[pl.ds(off, LANES)]
            ys = y_win[pl.ds(off, LANES)]
            zs = z_win[pl.ds(off, LANES)]
            hidx = idv * 4
            hx = plsc.load_gather(hbuf, [hidx])
            hy = plsc.load_gather(hbuf, [hidx + 1])
            hz = plsc.load_gather(hbuf, [hidx + 2])
            hw = plsc.load_gather(hbuf, [hidx + 3])
            s2p = xs * xs + ys * ys + zs * zs
            dot = xs * hx + ys * hy + zs * hz
            logit = av * s2p + dot + hw
            sig = 1.0 / (1.0 + jnp.exp(-logit))
            out_win[pl.ds(off, LANES)] = sig + 1e-4
            return 0
        lax.fori_loop(0, WIN // LANES, pt_body, 0)

        pltpu.sync_copy(out_win, out_hbm.at[pl.ds(base, WIN)])
        return 0
    lax.fori_loop(0, nwin, win_body, 0)


def kernel(pos, batch, W, b):
    n = pos.shape[0]
    w00 = jnp.full((LANES,), W[0, 0], jnp.float32)
    c0 = jnp.full((LANES,), W[0, 1] + b[0], jnp.float32)
    params = jnp.concatenate([w00, c0])

    xs = pos[:, 0]
    ys = pos[:, 1]
    zs = pos[:, 2]

    mesh = plsc.VectorSubcoreMesh(core_axis_name="c", subcore_axis_name="s")
    cparams = pltpu.CompilerParams(needs_layout_passes=False,
                                   use_tc_tiling_on_sc=False)

    k1 = functools.partial(
        pl.kernel,
        out_type=jax.ShapeDtypeStruct((NCORE * 4 * NSEG,), jnp.float32),
        mesh=mesh,
        compiler_params=cparams,
        scratch_types=[
            pltpu.VMEM((WIN,), jnp.float32),
            pltpu.VMEM((WIN,), jnp.float32),
            pltpu.VMEM((WIN,), jnp.float32),
            pltpu.VMEM((WIN,), jnp.int32),
            pltpu.VMEM((NSEG * LANES,), jnp.float32),
            pltpu.VMEM((4 * NSEG + 16,), jnp.float32),
            pltpu.VMEM_SHARED((NSUB, 4 * NSEG), jnp.float32),
        ],
    )(functools.partial(_phase1_body, n))
    part = k1(xs, ys, zs, batch)

    k2 = functools.partial(
        pl.kernel,
        out_type=jax.ShapeDtypeStruct((n,), jnp.float32),
        mesh=mesh,
        compiler_params=cparams,
        scratch_types=[
            pltpu.VMEM((WIN,), jnp.float32),
            pltpu.VMEM((WIN,), jnp.float32),
            pltpu.VMEM((WIN,), jnp.float32),
            pltpu.VMEM((WIN,), jnp.int32),
            pltpu.VMEM((WIN,), jnp.float32),
            pltpu.VMEM((4 * NSEG,), jnp.float32),
            pltpu.VMEM((4 * NSEG,), jnp.float32),
            pltpu.VMEM((4 * NSEG + 16,), jnp.float32),
            pltpu.VMEM((2 * LANES,), jnp.float32),
        ],
    )(functools.partial(_phase2_body, n))
    return k2(xs, ys, zs, batch, part, params)
